# baseline (device time: 249779 ns/iter reference)
import jax
import jax.numpy as jnp
from jax import lax
from jax.experimental import pallas as pl
from jax.experimental.pallas import tpu as pltpu

N_DEV = 32
SQ = 1024
SKV = 1024
H_PER = 8
DH = 128
D_MODEL = 1024
HD = H_PER * DH
CHUNK = SQ // N_DEV
SCALE = 0.08838834764831843
WINDOW = 128


def kernel(x, Wq, K_ext, V_ext, Wo):
    me = lax.axis_index("i")
    Wq_sl = lax.dynamic_slice(Wq, (0, me * HD), (D_MODEL, HD))
    Wo_sl = lax.dynamic_slice(Wo, (me * HD, 0), (HD, D_MODEL))
    x2 = x[0]
    K = K_ext[0]
    V = V_ext[0]

    def body(x_ref, wq_ref, k_ref, v_ref, wo_ref, out_ref,
             rs_buf, rs_send, rs_recv, ag_send, ag_recv):
        my = lax.axis_index("i")
        right = lax.rem(my + 1, N_DEV)
        left = lax.rem(my + N_DEV - 1, N_DEV)

        barrier = pltpu.get_barrier_semaphore()
        for nbr in (left, right):
            pl.semaphore_signal(barrier, inc=1, device_id=(nbr,),
                                device_id_type=pl.DeviceIdType.MESH)
        pl.semaphore_wait(barrier, 2)

        xv = x_ref[...]
        qi = lax.broadcasted_iota(jnp.int32, (SQ, SKV), 0)
        ki = lax.broadcasted_iota(jnp.int32, (SQ, SKV), 1)
        mask = jnp.abs(qi - ki) <= WINDOW
        for h in range(H_PER):
            q = jnp.dot(xv, wq_ref[:, h * DH:(h + 1) * DH],
                        preferred_element_type=jnp.float32)
            kh = k_ref[:, h, :]
            vh = v_ref[:, h, :]
            s = lax.dot_general(q, kh, (((1,), (1,)), ((), ())),
                                preferred_element_type=jnp.float32) * SCALE
            s = jnp.where(mask, s, jnp.float32(-1e9))
            m = jnp.max(s, axis=1, keepdims=True)
            p = jnp.exp(s - m)
            w = p / jnp.sum(p, axis=1, keepdims=True)
            ctx = jnp.dot(w, vh, preferred_element_type=jnp.float32)
            contrib = jnp.dot(ctx, wo_ref[h * DH:(h + 1) * DH, :],
                              preferred_element_type=jnp.float32)
            if h == 0:
                out_ref[...] = contrib
            else:
                out_ref[...] += contrib

        def rs_step(st, _):
            j_send = lax.rem(my + N_DEV - st, N_DEV)
            j_recv = lax.rem(my + 2 * N_DEV - st - 1, N_DEV)
            rdma = pltpu.make_async_remote_copy(
                src_ref=out_ref.at[pl.ds(j_send * CHUNK, CHUNK), :],
                dst_ref=rs_buf.at[st],
                send_sem=rs_send.at[st],
                recv_sem=rs_recv.at[st],
                device_id=(right,),
                device_id_type=pl.DeviceIdType.MESH,
            )
            rdma.start()
            rdma.wait()
            out_ref[pl.ds(j_recv * CHUNK, CHUNK), :] += rs_buf[st]
            return 0

        lax.fori_loop(0, N_DEV - 1, rs_step, 0)

        def ag_step(st, _):
            j_send = lax.rem(my + 2 * N_DEV + 1 - st, N_DEV)
            j_recv = lax.rem(my + 2 * N_DEV - st, N_DEV)
            send_d = pltpu.make_async_remote_copy(
                src_ref=out_ref.at[pl.ds(j_send * CHUNK, CHUNK), :],
                dst_ref=out_ref.at[pl.ds(j_send * CHUNK, CHUNK), :],
                send_sem=ag_send.at[st],
                recv_sem=ag_recv.at[st],
                device_id=(right,),
                device_id_type=pl.DeviceIdType.MESH,
            )
            send_d.start()
            recv_d = pltpu.make_async_remote_copy(
                src_ref=out_ref.at[pl.ds(j_recv * CHUNK, CHUNK), :],
                dst_ref=out_ref.at[pl.ds(j_recv * CHUNK, CHUNK), :],
                send_sem=ag_send.at[st],
                recv_sem=ag_recv.at[st],
                device_id=(left,),
                device_id_type=pl.DeviceIdType.MESH,
            )
            recv_d.wait_recv()
            send_d.wait_send()
            return 0

        lax.fori_loop(0, N_DEV - 1, ag_step, 0)

    out = pl.pallas_call(
        body,
        out_shape=jax.ShapeDtypeStruct((SQ, D_MODEL), jnp.float32),
        in_specs=[pl.BlockSpec(memory_space=pltpu.VMEM)] * 5,
        out_specs=pl.BlockSpec(memory_space=pltpu.VMEM),
        scratch_shapes=[
            pltpu.VMEM((N_DEV - 1, CHUNK, D_MODEL), jnp.float32),
            pltpu.SemaphoreType.DMA((N_DEV - 1,)),
            pltpu.SemaphoreType.DMA((N_DEV - 1,)),
            pltpu.SemaphoreType.DMA((N_DEV - 1,)),
            pltpu.SemaphoreType.DMA((N_DEV - 1,)),
        ],
        compiler_params=pltpu.CompilerParams(collective_id=0),
    )(x2, Wq_sl, K, V, Wo_sl)
    return out[None]
